# Initial kernel scaffold; baseline (speedup 1.0000x reference)
import jax
import jax.numpy as jnp
from jax.experimental import pallas as pl


def kernel(x, edge_index, W_self0, W_neigh0, b0, W_self1, W_neigh1, b1, W_self2, W_neigh2, b2):
    raise NotImplementedError("write your pallas kernel here")


# speedup vs baseline: 4.2366x; 4.2366x over previous
import functools

import jax
import jax.numpy as jnp
from jax import lax
from jax.experimental import pallas as pl
from jax.experimental.pallas import tpu as pltpu
from jax.experimental.pallas import tpu_sc as plsc

_N = 10000
_E = 320000
_D = 128
_C = 47

_NC = 2
_NS = 16
_NW = _NC * _NS
_EPW = _E // _NW
_CH = 80
_ITERS = _EPW // _CH
_NP = 10112
_RPT = _NP // _NS
_DEGW = 16

_mesh = plsc.VectorSubcoreMesh(
    core_axis_name="c", subcore_axis_name="s", num_cores=_NC, num_subcores=_NS
)


def _wid(cid, sid):
    return sid * _NC + cid


@functools.partial(
    pl.kernel,
    out_type=jax.ShapeDtypeStruct((_NC, _NP, _D), jnp.float32),
    mesh=_mesh,
    scratch_types=[
        pltpu.VMEM((_CH,), jnp.int32),
        pltpu.VMEM((_CH, _D), jnp.float32),
        pltpu.VMEM_SHARED((_NP, _D), jnp.float32),
    ],
)
def _sc_deg(dst_hbm, zeros_hbm, ones_hbm, out_hbm, dst_v, ones_v, deg_sh):
    cid = lax.axis_index("c")
    sid = lax.axis_index("s")
    r0 = sid * _RPT
    pltpu.sync_copy(zeros_hbm.at[pl.ds(r0, _RPT)], deg_sh.at[pl.ds(r0, _RPT)])
    pltpu.sync_copy(ones_hbm, ones_v)
    plsc.subcore_barrier()

    base = _wid(cid, sid) * _EPW

    def step(i, carry):
        off = base + i * _CH
        pltpu.sync_copy(dst_hbm.at[pl.ds(off, _CH)], dst_v)
        pltpu.sync_copy(ones_v, deg_sh.at[dst_v], add=True)
        return carry

    lax.fori_loop(0, _ITERS, step, 0)
    plsc.subcore_barrier()
    pltpu.sync_copy(deg_sh.at[pl.ds(r0, _RPT)], out_hbm.at[cid, pl.ds(r0, _RPT)])


@functools.partial(
    pl.kernel,
    out_type=jax.ShapeDtypeStruct((_NC, _NP, _D), jnp.float32),
    mesh=_mesh,
    scratch_types=[
        pltpu.VMEM((_CH,), jnp.int32),
        pltpu.VMEM((_CH,), jnp.int32),
        pltpu.VMEM((_CH, _D), jnp.float32),
        pltpu.VMEM_SHARED((_NP, _D), jnp.float32),
        pltpu.SemaphoreType.DMA,
    ],
)
def _sc_agg(h_hbm, src_hbm, dst_hbm, zeros_hbm, out_hbm,
            src_v, dst_v, rows_v, acc_sh, sem):
    cid = lax.axis_index("c")
    sid = lax.axis_index("s")
    r0 = sid * _RPT
    pltpu.sync_copy(zeros_hbm.at[pl.ds(r0, _RPT)], acc_sh.at[pl.ds(r0, _RPT)])
    plsc.subcore_barrier()

    base = _wid(cid, sid) * _EPW

    def step(i, carry):
        off = base + i * _CH
        pltpu.sync_copy(src_hbm.at[pl.ds(off, _CH)], src_v)
        pltpu.sync_copy(dst_hbm.at[pl.ds(off, _CH)], dst_v)
        pltpu.async_copy(h_hbm.at[src_v], rows_v, sem).wait()
        pltpu.sync_copy(rows_v, acc_sh.at[dst_v], add=True)
        return carry

    lax.fori_loop(0, _ITERS, step, 0)
    plsc.subcore_barrier()
    pltpu.sync_copy(acc_sh.at[pl.ds(r0, _RPT)], out_hbm.at[cid, pl.ds(r0, _RPT)])


_BN = 1000


def _tc_body(relu, h_b, a0_b, a1_b, d0_b, d1_b, ws_b, wn_b, b_b, o_b):
    deg = jnp.maximum(d0_b[:, 0:1] + d1_b[:, 0:1], 1.0)
    hn = (a0_b[...] + a1_b[...]) / deg
    o = (
        jnp.dot(h_b[...], ws_b[...], preferred_element_type=jnp.float32)
        + jnp.dot(hn, wn_b[...], preferred_element_type=jnp.float32)
        + b_b[...]
    )
    if relu:
        o = jnp.maximum(o, 0.0)
    o_b[...] = o


def _tc_layer(h, a0, a1, d0, d1, ws, wn, b, relu):
    body = functools.partial(_tc_body, relu)
    return pl.pallas_call(
        body,
        grid=(_N // _BN,),
        in_specs=[
            pl.BlockSpec((_BN, _D), lambda i: (i, 0)),
            pl.BlockSpec((_BN, _D), lambda i: (i, 0)),
            pl.BlockSpec((_BN, _D), lambda i: (i, 0)),
            pl.BlockSpec((_BN, _DEGW), lambda i: (i, 0)),
            pl.BlockSpec((_BN, _DEGW), lambda i: (i, 0)),
            pl.BlockSpec((_D, _D), lambda i: (0, 0)),
            pl.BlockSpec((_D, _D), lambda i: (0, 0)),
            pl.BlockSpec((1, _D), lambda i: (0, 0)),
        ],
        out_specs=pl.BlockSpec((_BN, _D), lambda i: (i, 0)),
        out_shape=jax.ShapeDtypeStruct((_N, _D), jnp.float32),
    )(h, a0, a1, d0, d1, ws, wn, b.reshape(1, _D))


def _pad_cols(w):
    return jnp.pad(w, ((0, 0), (0, _D - w.shape[1])))


def kernel(x, edge_index, W_self0, W_neigh0, b0, W_self1, W_neigh1, b1,
           W_self2, W_neigh2, b2):
    src = edge_index[0]
    dst = edge_index[1]
    z_nd = jnp.zeros((_NP, _D), jnp.float32)
    ones = jnp.ones((_CH, _D), jnp.float32)

    degp = _sc_deg(dst, z_nd, ones)
    d0, d1 = degp[0, :_N, :_DEGW], degp[1, :_N, :_DEGW]

    a = _sc_agg(x, src, dst, z_nd)
    h1 = _tc_layer(x, a[0, :_N], a[1, :_N], d0, d1, W_self0, W_neigh0, b0, relu=True)

    a = _sc_agg(h1, src, dst, z_nd)
    h2 = _tc_layer(h1, a[0, :_N], a[1, :_N], d0, d1, W_self1, W_neigh1, b1, relu=True)

    a = _sc_agg(h2, src, dst, z_nd)
    out = _tc_layer(
        h2, a[0, :_N], a[1, :_N], d0, d1,
        _pad_cols(W_self2), _pad_cols(W_neigh2),
        jnp.pad(b2, (0, _D - _C)), relu=False,
    )
    return out[:, :_C]
